# baseline (speedup 1.0000x reference)
import functools

import jax
import jax.numpy as jnp
from jax import lax
from jax.experimental import pallas as pl
from jax.experimental.pallas import tpu as pltpu
from jax.experimental.pallas import tpu_sc as plsc

_DIM = 128
_FLAT = _DIM * _DIM


def _softplus_kernel(x_ref, o_ref):
    o_ref[...] = jax.nn.softplus(x_ref[...])


def _tc_softplus(x):
    batch, dim = x.shape
    return pl.pallas_call(
        _softplus_kernel,
        grid=(1,),
        in_specs=[pl.BlockSpec((batch, dim), lambda i: (0, 0))],
        out_specs=pl.BlockSpec((batch, dim), lambda i: (0, 0)),
        out_shape=jax.ShapeDtypeStruct((batch, dim), x.dtype),
    )(x)


def _sc_diag_write(d):
    batch, dim = d.shape
    info = plsc.get_sparse_core_info()
    nw = info.num_cores * info.num_subcores
    per_w = batch // nw
    mesh = plsc.VectorSubcoreMesh(core_axis_name="c", subcore_axis_name="s")

    @functools.partial(
        pl.kernel,
        mesh=mesh,
        out_type=jax.ShapeDtypeStruct((batch, _FLAT), jnp.float32),
        scratch_types=[
            pltpu.VMEM((_FLAT,), jnp.float32),
            pltpu.VMEM((dim,), jnp.float32),
        ],
        compiler_params=pltpu.CompilerParams(
            use_tc_tiling_on_sc=False, needs_layout_passes=False
        ),
    )
    def k(d_hbm, out_hbm, buf, dbuf):
        wid = lax.axis_index("s") * info.num_cores + lax.axis_index("c")
        base = wid * per_w
        z16 = jnp.zeros((16,), jnp.float32)

        def zero_body(i, _):
            buf[pl.ds(i * 16, 16)] = z16
            return 0

        lax.fori_loop(0, _FLAT // 16, zero_body, 0)

        def elem_body(b, _):
            pltpu.sync_copy(d_hbm.at[base + b], dbuf)
            for j in range(dim // 16):
                idx = lax.iota(jnp.int32, 16) * (_DIM + 1) + j * 16 * (_DIM + 1)
                vals = dbuf[pl.ds(16 * j, 16)]
                plsc.store_scatter(buf, [idx], vals)
            pltpu.sync_copy(buf, out_hbm.at[base + b])
            for j in range(dim // 16):
                idx = lax.iota(jnp.int32, 16) * (_DIM + 1) + j * 16 * (_DIM + 1)
                plsc.store_scatter(buf, [idx], z16)
            return 0

        lax.fori_loop(0, per_w, elem_body, 0)

    return k(d)


def kernel(embeddings):
    batch, dim, _ = embeddings.shape
    x = embeddings[:, :, 0]
    d = _tc_softplus(x)
    flat = _sc_diag_write(d)
    return flat.reshape(batch, dim, dim)


# speedup vs baseline: 2.2720x; 2.2720x over previous
import jax
import jax.numpy as jnp
from jax.experimental import pallas as pl

_B_BLK = 128


def _diag_cov_kernel(x_ref, o_ref):
    d = jax.nn.softplus(x_ref[...])
    dim = d.shape[1]
    row = jax.lax.broadcasted_iota(jnp.int32, (dim, dim), 0)
    col = jax.lax.broadcasted_iota(jnp.int32, (dim, dim), 1)
    mask = row == col
    for b in range(d.shape[0]):
        o_ref[b, :, :] = jnp.where(mask, d[b][None, :], jnp.float32(0.0))


def kernel(embeddings):
    batch, dim, _ = embeddings.shape
    x = embeddings[:, :, 0]
    grid = (batch // _B_BLK,)
    return pl.pallas_call(
        _diag_cov_kernel,
        grid=grid,
        in_specs=[pl.BlockSpec((_B_BLK, dim), lambda i: (i, 0))],
        out_specs=pl.BlockSpec((_B_BLK, dim, dim), lambda i: (i, 0, 0)),
        out_shape=jax.ShapeDtypeStruct((batch, dim, dim), embeddings.dtype),
    )(x)
